# baseline (speedup 1.0000x reference)
import functools

import jax
import jax.numpy as jnp
from jax import lax
from jax.experimental import pallas as pl
from jax.experimental.pallas import tpu as pltpu
from jax.experimental.pallas import tpu_sc as plsc

N_ROWS = 1_000_000
K = 64
B = 16384

_UNROLL = 8


def _build():
    info = plsc.get_sparse_core_info()
    nc, ns = info.num_cores, info.num_subcores
    nw = nc * ns
    b_per_w = B // nw
    mesh = plsc.VectorSubcoreMesh(core_axis_name="c", subcore_axis_name="s")

    @functools.partial(
        pl.kernel,
        mesh=mesh,
        out_type=(
            jax.ShapeDtypeStruct((B, K), jnp.float32),
            jax.ShapeDtypeStruct((B, K), jnp.float32),
        ),
        scratch_types=[
            pltpu.VMEM_SHARED((ns, b_per_w), jnp.int32),
            pltpu.VMEM_SHARED((ns, b_per_w, K), jnp.float32),
            pltpu.SMEM((b_per_w,), jnp.int32),
            pltpu.VMEM((b_per_w, K), jnp.float32),
            pltpu.SemaphoreType.DMA,
            pltpu.SemaphoreType.DMA,
        ],
        compiler_params=pltpu.CompilerParams(needs_layout_passes=False),
    )
    def k(idx_hbm, mu_hbm, ls_hbm, mu_out, ls_out, idx_sh, ls_sh, idx_s, mu_v,
          sem_mu, sem_ls):
        cid = lax.axis_index("c")
        sid = lax.axis_index("s")
        wid = sid * nc + cid
        base = wid * b_per_w
        pltpu.sync_copy(idx_hbm.at[pl.ds(base, b_per_w)], idx_sh.at[sid])
        pltpu.sync_copy(idx_sh.at[sid], idx_s)

        def fire(g, _):
            for j in range(_UNROLL):
                i = g * _UNROLL + j
                r = idx_s[i]
                pltpu.async_copy(mu_hbm.at[r], mu_v.at[i], sem_mu)
                pltpu.async_copy(ls_hbm.at[r], ls_sh.at[sid, i], sem_ls)
            return _

        lax.fori_loop(0, b_per_w // _UNROLL, fire, None)

        def drain(i, _):
            pltpu.make_async_copy(mu_hbm.at[0], mu_v.at[0], sem_mu).wait()
            pltpu.make_async_copy(ls_hbm.at[0], ls_sh.at[sid, 0], sem_ls).wait()
            return _

        lax.fori_loop(0, b_per_w, drain, None)
        pltpu.sync_copy(mu_v, mu_out.at[pl.ds(base, b_per_w)])
        pltpu.sync_copy(ls_sh.at[sid], ls_out.at[pl.ds(base, b_per_w)])

    return k


_gather = _build()


def kernel(indices, mu, log_sigma):
    return _gather(indices.astype(jnp.int32), mu, log_sigma)
